# Initial kernel scaffold; baseline (speedup 1.0000x reference)
import jax
import jax.numpy as jnp
from jax.experimental import pallas as pl
from jax.experimental.pallas import tpu as pltpu

_B, _L, _H, _V, _K = 64, 8192, 64, 64, 8
_CHUNK = 512
_NSTEPS = _L // _CHUNK


def _dot(a, b):
    return jax.lax.dot_general(a, b, (((1,), (0,)), ((), ())),
                               preferred_element_type=jnp.float32)


def _dot_t(a, b):
    return jax.lax.dot_general(a, b, (((1,), (1,)), ((), ())),
                               preferred_element_type=jnp.float32)


def _tail(seq_blk, counts, refs):
    (embed_ref, w1_ref, b1_ref, w2_ref, b2_ref, gamma_ref, beta_ref,
     wg_ref, bg_ref, wq_ref, bq_ref, wo_ref, bo_ref,
     out_ref, mem_ref) = refs

    emb = embed_ref[...]
    t1 = jnp.maximum(_dot_t(emb, w1_ref[...]) + b1_ref[...], 0.0)
    ff = _dot_t(t1, w2_ref[...]) + b2_ref[...]
    x = emb + ff
    mu = jnp.mean(x, axis=1, keepdims=True)
    var = jnp.mean((x - mu) ** 2, axis=1, keepdims=True)
    ht = (x - mu) / jnp.sqrt(var + 1e-5) * gamma_ref[...] + beta_ref[...]

    wg = wg_ref[...]
    bg = bg_ref[0, 0]
    gt_col = _dot_t(ht, wg) + bg
    gt_row = _dot_t(wg, ht) + bg

    iota_row = jax.lax.broadcasted_iota(jnp.float32, (1, _V), 1)
    iota_col = jax.lax.broadcasted_iota(jnp.float32, (_V, 1), 0)

    m_uv = jnp.where(
        (gt_col > gt_row)
        | ((gt_col == gt_row) & (iota_col < iota_row)),
        1.0, 0.0)
    rank_row = jnp.sum(m_uv, axis=0, keepdims=True)
    m_vu = jnp.where(
        (gt_row > gt_col)
        | ((gt_row == gt_col) & (iota_row < iota_col)),
        1.0, 0.0)
    rank_col = jnp.sum(m_vu, axis=1, keepdims=True)

    p = jnp.where(rank_col == iota_row, 1.0, 0.0)
    pt = jnp.where(iota_col == rank_row, 1.0, 0.0)
    sc_sorted = _dot(counts, p)
    ht_sorted = _dot(pt, ht)

    tri = jnp.where(iota_col < iota_row, 1.0, 0.0)
    cum = _dot(sc_sorted, tri)
    upper = cum + sc_sorted

    last = seq_blk[:, _CHUNK - 1:_CHUNK]
    oh_last = jnp.where(
        last == jax.lax.broadcasted_iota(jnp.int32, (1, _V), 1), 1.0, 0.0)
    hl = _dot(oh_last, ht)
    q = _dot_t(hl, wq_ref[...]) + bq_ref[...]

    mems = []
    score_cols = []
    for k in range(_K):
        kf = jnp.float32(k)
        wk = jnp.where((cum <= kf) & (kf < upper), 1.0, 0.0)
        mem_k = _dot(wk, ht_sorted)
        mems.append(mem_k)
        mem_ref[pl.ds(k * _B, _B), :] = mem_k
        score_cols.append(jnp.sum(mem_k * q, axis=1, keepdims=True) * 0.125)
    scores = jnp.concatenate(score_cols, axis=1)
    smax = jnp.max(scores, axis=1, keepdims=True)
    ex = jnp.exp(scores - smax)
    attn = ex / jnp.sum(ex, axis=1, keepdims=True)

    ctx = jnp.zeros((_B, _H), jnp.float32)
    for k in range(_K):
        ctx = ctx + attn[:, k:k + 1] * mems[k]
    out_ref[...] = _dot_t(ctx, wo_ref[...]) + bo_ref[...]


def _kernel_body(seq_ref, embed_ref, w1_ref, b1_ref, w2_ref, b2_ref,
                 gamma_ref, beta_ref, wg_ref, bg_ref, wq_ref, bq_ref,
                 wo_ref, bo_ref, out_ref, mem_ref, counts_ref):
    step = pl.program_id(0)

    @pl.when(step == 0)
    def _init():
        counts_ref[...] = jnp.zeros((_B, _V), jnp.float32)

    seq_blk = seq_ref[...]
    cols = []
    for v in range(_V):
        cols.append(jnp.sum(jnp.where(seq_blk == v, 1.0, 0.0),
                            axis=1, keepdims=True))
    counts_ref[...] += jnp.concatenate(cols, axis=1)

    @pl.when(step == _NSTEPS - 1)
    def _finish():
        _tail(seq_blk, counts_ref[...],
              (embed_ref, w1_ref, b1_ref, w2_ref, b2_ref, gamma_ref,
               beta_ref, wg_ref, bg_ref, wq_ref, bq_ref, wo_ref, bo_ref,
               out_ref, mem_ref))


def kernel(seq, embed, W1, b1, W2, b2, gamma, beta, Wg, bg, Wq, bq, Wo, bo):
    full = lambda shape: pl.BlockSpec(shape, lambda i: (0, 0))
    out, mem_rows = pl.pallas_call(
        _kernel_body,
        grid=(_NSTEPS,),
        in_specs=[
            pl.BlockSpec((_B, _CHUNK), lambda i: (0, i)),
            full((_V, _H)),
            full((2 * _H, _H)),
            full((1, 2 * _H)),
            full((_H, 2 * _H)),
            full((1, _H)),
            full((1, _H)),
            full((1, _H)),
            full((1, _H)),
            full((1, 1)),
            full((_H, _H)),
            full((1, _H)),
            full((_V, _H)),
            full((1, _V)),
        ],
        out_specs=[
            pl.BlockSpec((_B, _V), lambda i: (0, 0)),
            pl.BlockSpec((_K * _B, _H), lambda i: (0, 0)),
        ],
        out_shape=[
            jax.ShapeDtypeStruct((_B, _V), jnp.float32),
            jax.ShapeDtypeStruct((_K * _B, _H), jnp.float32),
        ],
        scratch_shapes=[pltpu.VMEM((_B, _V), jnp.float32)],
    )(seq, embed, W1, b1.reshape(1, -1), W2, b2.reshape(1, -1),
      gamma.reshape(1, -1), beta.reshape(1, -1), Wg, bg.reshape(1, 1),
      Wq, bq.reshape(1, -1), Wo, bo.reshape(1, -1))
    memory = mem_rows.reshape(_K, _B, _H).transpose(1, 0, 2)
    return out, memory


# speedup vs baseline: 93.3477x; 93.3477x over previous
import jax
import jax.numpy as jnp
from jax.experimental import pallas as pl
from jax.experimental.pallas import tpu as pltpu

_B, _L, _H, _V, _K = 64, 8192, 64, 64, 8
_CHUNK = 512
_NSTEPS = _L // _CHUNK


def _dot(a, b):
    return jax.lax.dot_general(a, b, (((1,), (0,)), ((), ())),
                               preferred_element_type=jnp.float32)


def _dot_t(a, b):
    return jax.lax.dot_general(a, b, (((1,), (1,)), ((), ())),
                               preferred_element_type=jnp.float32)


def _dot0(a, b):
    return jax.lax.dot_general(a, b, (((0,), (0,)), ((), ())),
                               preferred_element_type=jnp.float32)


def _tail(seq_blk, counts, refs):
    (embed_ref, w1_ref, b1_ref, w2_ref, b2_ref, gamma_ref, beta_ref,
     wg_ref, bg_ref, wq_ref, bq_ref, wo_ref, bo_ref,
     out_ref, mem_ref) = refs

    emb = embed_ref[...]
    t1 = jnp.maximum(_dot_t(emb, w1_ref[...]) + b1_ref[...], 0.0)
    ff = _dot_t(t1, w2_ref[...]) + b2_ref[...]
    x = emb + ff
    mu = jnp.mean(x, axis=1, keepdims=True)
    var = jnp.mean((x - mu) ** 2, axis=1, keepdims=True)
    ht = (x - mu) / jnp.sqrt(var + 1e-5) * gamma_ref[...] + beta_ref[...]

    wgp = wg_ref[...]
    bg = bg_ref[0, 0]
    gt8 = _dot_t(ht, wgp) + bg

    iota_row = jax.lax.broadcasted_iota(jnp.int32, (1, _V), 1).astype(jnp.float32)
    iota_col = jax.lax.broadcasted_iota(jnp.int32, (_V, 1), 0).astype(jnp.float32)
    eye = jnp.where(iota_col == iota_row, 1.0, 0.0)

    def _transpose8(cols8):
        return _dot0(cols8, eye)

    gt_col = gt8[:, 0:1]
    gt_row = _transpose8(gt8)[0:1, :]

    m_vu = jnp.where(
        (gt_row > gt_col)
        | ((gt_row == gt_col) & (iota_row < iota_col)),
        1.0, 0.0)
    rank_col = jnp.sum(m_vu, axis=1, keepdims=True)
    rank_row = _transpose8(rank_col * jnp.ones((1, 8), jnp.float32))[0:1, :]

    p = jnp.where(rank_col == iota_row, 1.0, 0.0)
    pt = jnp.where(iota_col == rank_row, 1.0, 0.0)
    sc_sorted = _dot(counts, p)
    ht_sorted = _dot(pt, ht)

    tri = jnp.where(iota_col < iota_row, 1.0, 0.0)
    cum = _dot(sc_sorted, tri)
    upper = cum + sc_sorted

    last = seq_blk[:, _CHUNK - 1:_CHUNK]
    oh_last = jnp.where(
        last == jax.lax.broadcasted_iota(jnp.int32, (1, _V), 1), 1.0, 0.0)
    hl = _dot(oh_last, ht)
    q = _dot_t(hl, wq_ref[...]) + bq_ref[...]

    mems = []
    score_cols = []
    for k in range(_K):
        kf = jnp.float32(k)
        wk = jnp.where((cum <= kf) & (kf < upper), 1.0, 0.0)
        mem_k = _dot(wk, ht_sorted)
        mems.append(mem_k)
        mem_ref[pl.ds(k * _B, _B), :] = mem_k
        score_cols.append(jnp.sum(mem_k * q, axis=1, keepdims=True) * 0.125)
    scores = jnp.concatenate(score_cols, axis=1)
    smax = jnp.max(scores, axis=1, keepdims=True)
    ex = jnp.exp(scores - smax)
    attn = ex / jnp.sum(ex, axis=1, keepdims=True)

    ctx = jnp.zeros((_B, _H), jnp.float32)
    for k in range(_K):
        ctx = ctx + attn[:, k:k + 1] * mems[k]
    out_ref[...] = _dot_t(ctx, wo_ref[...]) + bo_ref[...]


def _kernel_body(seq_ref, embed_ref, w1_ref, b1_ref, w2_ref, b2_ref,
                 gamma_ref, beta_ref, wg_ref, bg_ref, wq_ref, bq_ref,
                 wo_ref, bo_ref, out_ref, mem_ref, counts_ref):
    step = pl.program_id(0)

    @pl.when(step == 0)
    def _init():
        counts_ref[...] = jnp.zeros((_B, _V), jnp.float32)

    seq_blk = seq_ref[...]
    cols = []
    for v in range(_V):
        cols.append(jnp.sum(jnp.where(seq_blk == v, 1.0, 0.0),
                            axis=1, keepdims=True))
    counts_ref[...] += jnp.concatenate(cols, axis=1)

    @pl.when(step == _NSTEPS - 1)
    def _finish():
        _tail(seq_blk, counts_ref[...],
              (embed_ref, w1_ref, b1_ref, w2_ref, b2_ref, gamma_ref,
               beta_ref, wg_ref, bg_ref, wq_ref, bq_ref, wo_ref, bo_ref,
               out_ref, mem_ref))


def kernel(seq, embed, W1, b1, W2, b2, gamma, beta, Wg, bg, Wq, bq, Wo, bo):
    full = lambda shape: pl.BlockSpec(shape, lambda i: (0, 0))
    out, mem_rows = pl.pallas_call(
        _kernel_body,
        grid=(_NSTEPS,),
        in_specs=[
            pl.BlockSpec((_B, _CHUNK), lambda i: (0, i)),
            full((_V, _H)),
            full((2 * _H, _H)),
            full((1, 2 * _H)),
            full((_H, 2 * _H)),
            full((1, _H)),
            full((1, _H)),
            full((1, _H)),
            full((8, _H)),
            full((1, 1)),
            full((_H, _H)),
            full((1, _H)),
            full((_V, _H)),
            full((1, _V)),
        ],
        out_specs=[
            pl.BlockSpec((_B, _V), lambda i: (0, 0)),
            pl.BlockSpec((_K * _B, _H), lambda i: (0, 0)),
        ],
        out_shape=[
            jax.ShapeDtypeStruct((_B, _V), jnp.float32),
            jax.ShapeDtypeStruct((_K * _B, _H), jnp.float32),
        ],
        scratch_shapes=[pltpu.VMEM((_B, _V), jnp.float32)],
    )(seq, embed, W1, b1.reshape(1, -1), W2, b2.reshape(1, -1),
      gamma.reshape(1, -1), beta.reshape(1, -1),
      jnp.concatenate([Wg, jnp.zeros((7, _H), Wg.dtype)], axis=0),
      bg.reshape(1, 1),
      Wq, bq.reshape(1, -1), Wo, bo.reshape(1, -1))
    memory = mem_rows.reshape(_K, _B, _H).transpose(1, 0, 2)
    return out, memory
